# baseline (device time: 12084 ns/iter reference)
import jax
import jax.numpy as jnp
from jax import lax
from jax.experimental import pallas as pl
from jax.experimental.pallas import tpu as pltpu

N_DEV = 8

_FAR = (6,)
_MID = (5, 7, 2)
_NEAR = (1, 3, 4)


def kernel(x):
    m, n = x.shape
    chunk = m // N_DEV

    def body(x_ref, out_ref, xbf, rs_bufs, mine,
             rs_send_sem, rs_near_sem, rs_mid_sem, rs_far_sem,
             ag_send_sem, ag_recv_sem):
        my = lax.axis_index("i")

        barrier_sem = pltpu.get_barrier_semaphore()
        for k in range(1, N_DEV):
            pl.semaphore_signal(
                barrier_sem, inc=1,
                device_id=((my + k) % N_DEV,),
                device_id_type=pl.DeviceIdType.MESH,
            )
        xbf[...] = x_ref[...].astype(jnp.bfloat16)
        rs_bufs[pl.ds(my * chunk, chunk), :] = xbf[pl.ds(my * chunk, chunk), :]
        pl.semaphore_wait(barrier_sem, N_DEV - 1)

        group_rdmas = {}
        for masks, sem in ((_FAR, rs_far_sem), (_MID, rs_mid_sem),
                           (_NEAR, rs_near_sem)):
            rdmas = []
            for mask in masks:
                dst = my ^ mask
                rdma = pltpu.make_async_remote_copy(
                    src_ref=xbf.at[pl.ds(dst * chunk, chunk), :],
                    dst_ref=rs_bufs.at[pl.ds(my * chunk, chunk), :],
                    send_sem=rs_send_sem,
                    recv_sem=sem,
                    device_id=(dst,),
                    device_id_type=pl.DeviceIdType.MESH,
                )
                rdma.start()
                rdmas.append(rdma)
            group_rdmas[masks] = rdmas

        acc = rs_bufs[pl.ds(my * chunk, chunk), :]
        for masks in (_NEAR, _MID, _FAR):
            for rdma in group_rdmas[masks]:
                rdma.wait_recv()
            for mask in masks:
                s = my ^ mask
                acc = acc + rs_bufs[pl.ds(s * chunk, chunk), :]
        mine[...] = acc

        ag_rdmas = []
        for mask in _FAR + _MID + _NEAR:
            dst = my ^ mask
            rdma = pltpu.make_async_remote_copy(
                src_ref=mine,
                dst_ref=out_ref.at[pl.ds(my * chunk, chunk), :],
                send_sem=ag_send_sem,
                recv_sem=ag_recv_sem,
                device_id=(dst,),
                device_id_type=pl.DeviceIdType.MESH,
            )
            rdma.start()
            ag_rdmas.append(rdma)
        out_ref[pl.ds(my * chunk, chunk), :] = mine[...]
        for rdma in ag_rdmas:
            rdma.wait_recv()

        for rdmas in group_rdmas.values():
            for rdma in rdmas:
                rdma.wait_send()
        for rdma in ag_rdmas:
            rdma.wait_send()

    return pl.pallas_call(
        body,
        out_shape=jax.ShapeDtypeStruct((m, n), jnp.bfloat16),
        in_specs=[pl.BlockSpec(memory_space=pltpu.VMEM)],
        out_specs=pl.BlockSpec(memory_space=pltpu.VMEM),
        scratch_shapes=[
            pltpu.VMEM((m, n), jnp.bfloat16),
            pltpu.VMEM((N_DEV * chunk, n), jnp.bfloat16),
            pltpu.VMEM((chunk, n), jnp.bfloat16),
            pltpu.SemaphoreType.DMA,
            pltpu.SemaphoreType.DMA,
            pltpu.SemaphoreType.DMA,
            pltpu.SemaphoreType.DMA,
            pltpu.SemaphoreType.DMA,
            pltpu.SemaphoreType.DMA,
        ],
        compiler_params=pltpu.CompilerParams(collective_id=0),
    )(x)
